# baseline (speedup 1.0000x reference)
import jax
import jax.numpy as jnp
from jax.experimental import pallas as pl
from jax.experimental.pallas import tpu as pltpu

_CHUNK = 1024
_NBUF = 2


def _dsq_kernel(sel_hbm, ms_ref, out_ref, buf, sem):
    k = ms_ref.shape[1] - 1
    qs = ms_ref[:, :k] + ms_ref[:, k:k + 1]
    logqs = jnp.log(qs)
    b = out_ref.shape[0]
    nchunks = b // _CHUNK

    def copy_in(c, slot):
        return pltpu.make_async_copy(
            sel_hbm.at[pl.ds(c * _CHUNK, _CHUNK), :], buf.at[slot], sem.at[slot])

    for c in range(_NBUF):
        copy_in(c, c).start()
    for c in range(nchunks):
        slot = c % _NBUF
        copy_in(c, slot).wait()
        fire = 1.0 - buf[slot].astype(jnp.float32)
        acc = jnp.dot(fire, logqs, preferred_element_type=jnp.float32)
        res = jnp.exp(acc)
        res = jnp.where(res <= 1e-16, res + 1e-16, res)
        out_ref[pl.ds(c * _CHUNK, _CHUNK), :] = (
            res / jnp.sum(res, axis=1, keepdims=True))
        if c + _NBUF < nchunks:
            copy_in(c + _NBUF, slot).start()


def kernel(X, ms, sel):
    b, n = sel.shape
    k = ms.shape[1] - 1
    return pl.pallas_call(
        _dsq_kernel,
        in_specs=[
            pl.BlockSpec(memory_space=pl.ANY),
            pl.BlockSpec((n, k + 1), lambda: (0, 0)),
        ],
        out_specs=pl.BlockSpec((b, k), lambda: (0, 0)),
        out_shape=jax.ShapeDtypeStruct((b, k), jnp.float32),
        scratch_shapes=[
            pltpu.VMEM((_NBUF, _CHUNK, n), jnp.int8),
            pltpu.SemaphoreType.DMA((_NBUF,)),
        ],
    )(sel.view(jnp.int8), ms)
